# Initial kernel scaffold; baseline (speedup 1.0000x reference)
import jax
import jax.numpy as jnp
from jax.experimental import pallas as pl


def kernel(x, edge_index, W1, b1, W2, b2):
    raise NotImplementedError("write your pallas kernel here")


# speedup vs baseline: 221.2561x; 221.2561x over previous
import functools

import jax
import jax.numpy as jnp
from jax import lax
from jax.experimental import pallas as pl
from jax.experimental.pallas import tpu as pltpu
from jax.experimental.pallas import tpu_sc as plsc

N_NODES = 100000
N_PAD = 100352
ROWS_N = 784
E_EDGES = 6400000
E_PAD = 6553600
E_ROWS = E_PAD // 128
WIN = 8192
WROWS = WIN // 128
NC, NS = 2, 16
NW = NC * NS

_mesh = plsc.VectorSubcoreMesh(
    core_axis_name="c", subcore_axis_name="s", num_cores=NC, num_subcores=NS)


@functools.partial(
    pl.kernel,
    out_type=jax.ShapeDtypeStruct((NC, N_PAD), jnp.float32),
    mesh=_mesh,
    scratch_types=[
        pltpu.VMEM((WIN,), jnp.int32),
        pltpu.VMEM((WIN,), jnp.float32),
        pltpu.VMEM_SHARED((N_PAD,), jnp.float32),
    ],
)
def _sc_count(dst_hbm, zeros_hbm, ones_hbm, out_hbm, idx_v, upd_v, acc_sh):
    cid = lax.axis_index("c")
    sid = lax.axis_index("s")
    wid = sid * NC + cid

    @pl.when(sid == 0)
    def _():
        pltpu.sync_copy(zeros_hbm, acc_sh)

    pltpu.sync_copy(ones_hbm, upd_v)
    plsc.subcore_barrier()

    def body(w, carry):
        e0 = wid * (E_PAD // NW) + w * WIN
        pltpu.sync_copy(dst_hbm.at[pl.ds(e0, WIN)], idx_v)
        pltpu.sync_copy(upd_v, acc_sh.at[idx_v], add=True)
        return carry

    lax.fori_loop(0, E_PAD // NW // WIN, body, 0)
    plsc.subcore_barrier()
    sl = N_PAD // NS
    pltpu.sync_copy(acc_sh.at[pl.ds(sid * sl, sl)],
                    out_hbm.at[cid, pl.ds(sid * sl, sl)])


@functools.partial(
    pl.kernel,
    out_type=jax.ShapeDtypeStruct((NC, N_PAD), jnp.float32),
    mesh=_mesh,
    scratch_types=[
        pltpu.VMEM((WIN,), jnp.int32),
        pltpu.VMEM((WIN,), jnp.int32),
        pltpu.VMEM((WIN,), jnp.float32),
        pltpu.VMEM_SHARED((N_PAD,), jnp.float32),
        pltpu.VMEM_SHARED((N_PAD,), jnp.float32),
        pltpu.SemaphoreType.DMA,
    ],
)
def _sc_seg_scalar(src_hbm, dst_hbm, tab_hbm, zeros_hbm, out_hbm,
                   sidx_v, didx_v, upd_v, tab_sh, acc_sh, sem):
    cid = lax.axis_index("c")
    sid = lax.axis_index("s")
    wid = sid * NC + cid

    @pl.when(sid == 0)
    def _():
        pltpu.sync_copy(zeros_hbm, acc_sh)

    @pl.when(sid == 1)
    def _():
        pltpu.sync_copy(tab_hbm, tab_sh)

    plsc.subcore_barrier()

    def body(w, carry):
        e0 = wid * (E_PAD // NW) + w * WIN
        pltpu.sync_copy(src_hbm.at[pl.ds(e0, WIN)], sidx_v)
        pltpu.sync_copy(dst_hbm.at[pl.ds(e0, WIN)], didx_v)
        pltpu.async_copy(tab_sh.at[sidx_v], upd_v, sem).wait()
        pltpu.sync_copy(upd_v, acc_sh.at[didx_v], add=True)
        return carry

    lax.fori_loop(0, E_PAD // NW // WIN, body, 0)
    plsc.subcore_barrier()
    sl = N_PAD // NS
    pltpu.sync_copy(acc_sh.at[pl.ds(sid * sl, sl)],
                    out_hbm.at[cid, pl.ds(sid * sl, sl)])


@functools.partial(
    pl.kernel,
    out_type=jax.ShapeDtypeStruct((NC, N_PAD), jnp.float32),
    mesh=_mesh,
    scratch_types=[
        pltpu.VMEM((WIN,), jnp.int32),
        pltpu.VMEM((WIN,), jnp.int32),
        pltpu.VMEM((WIN,), jnp.float32),
        pltpu.VMEM_SHARED((N_PAD,), jnp.float32),
        pltpu.VMEM_SHARED((N_PAD,), jnp.float32),
        pltpu.SemaphoreType.DMA,
    ],
)
def _sc_seg_feat(src_hbm, dst_hbm, qtab_hbm, zeros_hbm, out_hbm,
                 sidx_v, didx_v, upd_v, tab_sh, acc_sh, sem):
    cid = lax.axis_index("c")
    sid = lax.axis_index("s")

    @pl.when(sid == 0)
    def _():
        pltpu.sync_copy(zeros_hbm, acc_sh)

    @pl.when(sid == 1)
    def _():
        pltpu.sync_copy(qtab_hbm.at[cid], tab_sh)

    plsc.subcore_barrier()

    def body(w, carry):
        e0 = sid * (E_PAD // NS) + w * WIN
        pltpu.sync_copy(src_hbm.at[pl.ds(e0, WIN)], sidx_v)
        pltpu.sync_copy(dst_hbm.at[pl.ds(e0, WIN)], didx_v)
        pltpu.async_copy(tab_sh.at[sidx_v], upd_v, sem).wait()
        pltpu.sync_copy(upd_v, acc_sh.at[didx_v], add=True)
        return carry

    lax.fori_loop(0, E_PAD // NS // WIN, body, 0)
    plsc.subcore_barrier()
    sl = N_PAD // NS
    pltpu.sync_copy(acc_sh.at[pl.ds(sid * sl, sl)],
                    out_hbm.at[cid, pl.ds(sid * sl, sl)])


def _tc_prep_body(cnt0, cnt1, xr, dinv_o, p_o):
    deg = cnt0[...] + cnt1[...] + 1.0
    dinv = lax.rsqrt(deg)
    dinv_o[...] = dinv
    p_o[...] = dinv * xr[...]


_tc_prep = pl.pallas_call(
    _tc_prep_body,
    out_shape=[jax.ShapeDtypeStruct((ROWS_N, 128), jnp.float32)] * 2,
)


def _tc_mid_body(segp0, segp1, dinv_r, p_r, W1_r, b1_r, W2_r, q0_o, q1_o):
    dinv = dinv_r[...]
    s = dinv * (segp0[...] + segp1[...] + p_r[...])
    t0 = jnp.zeros_like(s)
    t1 = jnp.zeros_like(s)
    for j in range(16):
        h = jnp.maximum(s * W1_r[0, j] + b1_r[j], 0.0)
        t0 = t0 + h * W2_r[j, 0]
        t1 = t1 + h * W2_r[j, 1]
    q0_o[...] = dinv * t0
    q1_o[...] = dinv * t1


_tc_mid = pl.pallas_call(
    _tc_mid_body,
    in_specs=[pl.BlockSpec(memory_space=pltpu.VMEM)] * 4
    + [pl.BlockSpec(memory_space=pltpu.SMEM)] * 3,
    out_shape=[jax.ShapeDtypeStruct((ROWS_N, 128), jnp.float32)] * 2,
)


def _tc_final_body(segq0, segq1, dinv_r, q0_r, q1_r, b2_r, o0, o1):
    dinv = dinv_r[...]
    o0[...] = dinv * (segq0[...] + q0_r[...]) + b2_r[0]
    o1[...] = dinv * (segq1[...] + q1_r[...]) + b2_r[1]


_tc_final = pl.pallas_call(
    _tc_final_body,
    in_specs=[pl.BlockSpec(memory_space=pltpu.VMEM)] * 5
    + [pl.BlockSpec(memory_space=pltpu.SMEM)],
    out_shape=[jax.ShapeDtypeStruct((ROWS_N, 128), jnp.float32)] * 2,
)


def kernel(x, edge_index, W1, b1, W2, b2):
    src = edge_index[0].astype(jnp.int32)
    dst = edge_index[1].astype(jnp.int32)

    pad_n = E_PAD - E_EDGES
    padv = (N_NODES + jnp.arange(pad_n, dtype=jnp.int32) % (N_PAD - N_NODES))
    src_p = jnp.concatenate([src, padv])
    dst_p = jnp.concatenate([dst, padv])

    xpad = jnp.pad(x[:, 0], (0, N_PAD - N_NODES))
    zeros = jnp.zeros((N_PAD,), jnp.float32)
    ones = jnp.ones((WIN,), jnp.float32)

    cnt = _sc_count(dst_p, zeros, ones)
    dinv, p = _tc_prep(cnt[0].reshape(ROWS_N, 128),
                       cnt[1].reshape(ROWS_N, 128),
                       xpad.reshape(ROWS_N, 128))
    segp = _sc_seg_scalar(src_p, dst_p, p.reshape(N_PAD), zeros)
    q0, q1 = _tc_mid(segp[0].reshape(ROWS_N, 128),
                     segp[1].reshape(ROWS_N, 128),
                     dinv, p, W1, b1, W2)
    qtab = jnp.stack([q0.reshape(N_PAD), q1.reshape(N_PAD)])
    segq = _sc_seg_feat(src_p, dst_p, qtab, zeros)
    o0, o1 = _tc_final(segq[0].reshape(ROWS_N, 128),
                       segq[1].reshape(ROWS_N, 128),
                       dinv, q0, q1, b2)
    return jnp.stack([o0.reshape(N_PAD)[:N_NODES],
                      o1.reshape(N_PAD)[:N_NODES]], axis=1)


# speedup vs baseline: 237.1615x; 1.0719x over previous
import functools

import jax
import jax.numpy as jnp
from jax import lax
from jax.experimental import pallas as pl
from jax.experimental.pallas import tpu as pltpu
from jax.experimental.pallas import tpu_sc as plsc

N_NODES = 100000
N_PAD = 100352
ROWS_N = 784
E_EDGES = 6400000
E_PAD = 6553600
E_ROWS = E_PAD // 128
WIN = 8192
WROWS = WIN // 128
NC, NS = 2, 16
NW = NC * NS

_mesh = plsc.VectorSubcoreMesh(
    core_axis_name="c", subcore_axis_name="s", num_cores=NC, num_subcores=NS)


@functools.partial(
    pl.kernel,
    out_type=jax.ShapeDtypeStruct((NC, N_PAD), jnp.float32),
    mesh=_mesh,
    scratch_types=[
        pltpu.VMEM((WIN,), jnp.int32),
        pltpu.VMEM((WIN,), jnp.float32),
        pltpu.VMEM_SHARED((N_PAD,), jnp.float32),
    ],
)
def _sc_count(dst_hbm, zeros_hbm, ones_hbm, out_hbm, idx_v, upd_v, acc_sh):
    cid = lax.axis_index("c")
    sid = lax.axis_index("s")
    wid = sid * NC + cid

    @pl.when(sid == 0)
    def _():
        pltpu.sync_copy(zeros_hbm, acc_sh)

    pltpu.sync_copy(ones_hbm, upd_v)
    plsc.subcore_barrier()

    def body(w, carry):
        e0 = wid * (E_PAD // NW) + w * WIN
        pltpu.sync_copy(dst_hbm.at[pl.ds(e0, WIN)], idx_v)
        pltpu.sync_copy(upd_v, acc_sh.at[idx_v], add=True)
        return carry

    lax.fori_loop(0, E_PAD // NW // WIN, body, 0)
    plsc.subcore_barrier()
    sl = N_PAD // NS
    pltpu.sync_copy(acc_sh.at[pl.ds(sid * sl, sl)],
                    out_hbm.at[cid, pl.ds(sid * sl, sl)])


@functools.partial(
    pl.kernel,
    out_type=jax.ShapeDtypeStruct((NC, N_PAD), jnp.float32),
    mesh=_mesh,
    scratch_types=[
        pltpu.VMEM((WIN,), jnp.int32),
        pltpu.VMEM((WIN,), jnp.int32),
        pltpu.VMEM((WIN,), jnp.float32),
        pltpu.VMEM_SHARED((N_PAD,), jnp.float32),
        pltpu.VMEM_SHARED((N_PAD,), jnp.float32),
        pltpu.SemaphoreType.DMA,
    ],
)
def _sc_seg_scalar(src_hbm, dst_hbm, tab_hbm, zeros_hbm, out_hbm,
                   sidx_v, didx_v, upd_v, tab_sh, acc_sh, sem):
    cid = lax.axis_index("c")
    sid = lax.axis_index("s")
    wid = sid * NC + cid

    @pl.when(sid == 0)
    def _():
        pltpu.sync_copy(zeros_hbm, acc_sh)

    @pl.when(sid == 1)
    def _():
        pltpu.sync_copy(tab_hbm, tab_sh)

    plsc.subcore_barrier()

    def body(w, carry):
        e0 = wid * (E_PAD // NW) + w * WIN
        pltpu.sync_copy(src_hbm.at[pl.ds(e0, WIN)], sidx_v)
        pltpu.sync_copy(dst_hbm.at[pl.ds(e0, WIN)], didx_v)
        pltpu.async_copy(tab_sh.at[sidx_v], upd_v, sem).wait()
        pltpu.sync_copy(upd_v, acc_sh.at[didx_v], add=True)
        return carry

    lax.fori_loop(0, E_PAD // NW // WIN, body, 0)
    plsc.subcore_barrier()
    sl = N_PAD // NS
    pltpu.sync_copy(acc_sh.at[pl.ds(sid * sl, sl)],
                    out_hbm.at[cid, pl.ds(sid * sl, sl)])


@functools.partial(
    pl.kernel,
    out_type=jax.ShapeDtypeStruct((NC, 2, N_PAD), jnp.float32),
    mesh=_mesh,
    scratch_types=[
        pltpu.VMEM((WIN,), jnp.int32),
        pltpu.VMEM((WIN,), jnp.int32),
        pltpu.VMEM((WIN,), jnp.float32),
        pltpu.VMEM((WIN,), jnp.float32),
        pltpu.VMEM_SHARED((N_PAD,), jnp.float32),
        pltpu.VMEM_SHARED((N_PAD,), jnp.float32),
        pltpu.VMEM_SHARED((N_PAD,), jnp.float32),
        pltpu.VMEM_SHARED((N_PAD,), jnp.float32),
        pltpu.SemaphoreType.DMA,
    ],
)
def _sc_seg_feat(src_hbm, dst_hbm, qtab_hbm, zeros_hbm, out_hbm,
                 sidx_v, didx_v, upd0_v, upd1_v,
                 tab0_sh, tab1_sh, acc0_sh, acc1_sh, sem):
    cid = lax.axis_index("c")
    sid = lax.axis_index("s")
    wid = sid * NC + cid

    @pl.when(sid == 0)
    def _():
        pltpu.sync_copy(zeros_hbm, acc0_sh)

    @pl.when(sid == 1)
    def _():
        pltpu.sync_copy(zeros_hbm, acc1_sh)

    @pl.when(sid == 2)
    def _():
        pltpu.sync_copy(qtab_hbm.at[0], tab0_sh)

    @pl.when(sid == 3)
    def _():
        pltpu.sync_copy(qtab_hbm.at[1], tab1_sh)

    plsc.subcore_barrier()

    def body(w, carry):
        e0 = wid * (E_PAD // NW) + w * WIN
        pltpu.sync_copy(src_hbm.at[pl.ds(e0, WIN)], sidx_v)
        pltpu.sync_copy(dst_hbm.at[pl.ds(e0, WIN)], didx_v)
        pltpu.async_copy(tab0_sh.at[sidx_v], upd0_v, sem).wait()
        pltpu.async_copy(tab1_sh.at[sidx_v], upd1_v, sem).wait()
        pltpu.sync_copy(upd0_v, acc0_sh.at[didx_v], add=True)
        pltpu.sync_copy(upd1_v, acc1_sh.at[didx_v], add=True)
        return carry

    lax.fori_loop(0, E_PAD // NW // WIN, body, 0)
    plsc.subcore_barrier()
    sl = N_PAD // NS
    pltpu.sync_copy(acc0_sh.at[pl.ds(sid * sl, sl)],
                    out_hbm.at[cid, 0, pl.ds(sid * sl, sl)])
    pltpu.sync_copy(acc1_sh.at[pl.ds(sid * sl, sl)],
                    out_hbm.at[cid, 1, pl.ds(sid * sl, sl)])


def _tc_prep_body(cnt0, cnt1, xr, dinv_o, p_o):
    deg = cnt0[...] + cnt1[...] + 1.0
    dinv = lax.rsqrt(deg)
    dinv_o[...] = dinv
    p_o[...] = dinv * xr[...]


_tc_prep = pl.pallas_call(
    _tc_prep_body,
    out_shape=[jax.ShapeDtypeStruct((ROWS_N, 128), jnp.float32)] * 2,
)


def _tc_mid_body(segp0, segp1, dinv_r, p_r, W1_r, b1_r, W2_r, q0_o, q1_o):
    dinv = dinv_r[...]
    s = dinv * (segp0[...] + segp1[...] + p_r[...])
    t0 = jnp.zeros_like(s)
    t1 = jnp.zeros_like(s)
    for j in range(16):
        h = jnp.maximum(s * W1_r[0, j] + b1_r[j], 0.0)
        t0 = t0 + h * W2_r[j, 0]
        t1 = t1 + h * W2_r[j, 1]
    q0_o[...] = dinv * t0
    q1_o[...] = dinv * t1


_tc_mid = pl.pallas_call(
    _tc_mid_body,
    in_specs=[pl.BlockSpec(memory_space=pltpu.VMEM)] * 4
    + [pl.BlockSpec(memory_space=pltpu.SMEM)] * 3,
    out_shape=[jax.ShapeDtypeStruct((ROWS_N, 128), jnp.float32)] * 2,
)


def _tc_final_body(sq00, sq01, sq10, sq11, dinv_r, q0_r, q1_r, b2_r, o0, o1):
    dinv = dinv_r[...]
    o0[...] = dinv * (sq00[...] + sq10[...] + q0_r[...]) + b2_r[0]
    o1[...] = dinv * (sq01[...] + sq11[...] + q1_r[...]) + b2_r[1]


_tc_final = pl.pallas_call(
    _tc_final_body,
    in_specs=[pl.BlockSpec(memory_space=pltpu.VMEM)] * 7
    + [pl.BlockSpec(memory_space=pltpu.SMEM)],
    out_shape=[jax.ShapeDtypeStruct((ROWS_N, 128), jnp.float32)] * 2,
)


def kernel(x, edge_index, W1, b1, W2, b2):
    src = edge_index[0].astype(jnp.int32)
    dst = edge_index[1].astype(jnp.int32)

    pad_n = E_PAD - E_EDGES
    padv = (N_NODES + jnp.arange(pad_n, dtype=jnp.int32) % (N_PAD - N_NODES))
    src_p = jnp.concatenate([src, padv])
    dst_p = jnp.concatenate([dst, padv])

    xpad = jnp.pad(x[:, 0], (0, N_PAD - N_NODES))
    zeros = jnp.zeros((N_PAD,), jnp.float32)
    ones = jnp.ones((WIN,), jnp.float32)

    cnt = _sc_count(dst_p, zeros, ones)
    dinv, p = _tc_prep(cnt[0].reshape(ROWS_N, 128),
                       cnt[1].reshape(ROWS_N, 128),
                       xpad.reshape(ROWS_N, 128))
    segp = _sc_seg_scalar(src_p, dst_p, p.reshape(N_PAD), zeros)
    q0, q1 = _tc_mid(segp[0].reshape(ROWS_N, 128),
                     segp[1].reshape(ROWS_N, 128),
                     dinv, p, W1, b1, W2)
    qtab = jnp.stack([q0.reshape(N_PAD), q1.reshape(N_PAD)])
    segq = _sc_seg_feat(src_p, dst_p, qtab, zeros)
    o0, o1 = _tc_final(segq[0, 0].reshape(ROWS_N, 128),
                       segq[0, 1].reshape(ROWS_N, 128),
                       segq[1, 0].reshape(ROWS_N, 128),
                       segq[1, 1].reshape(ROWS_N, 128),
                       dinv, q0, q1, b2)
    return jnp.stack([o0.reshape(N_PAD)[:N_NODES],
                      o1.reshape(N_PAD)[:N_NODES]], axis=1)
